# baseline (speedup 1.0000x reference)
import functools

import jax
import jax.numpy as jnp
from jax.experimental import pallas as pl
from jax.experimental.pallas import tpu as pltpu

_DEPTH = 6


def _silu(x):
    return x * jax.nn.sigmoid(x)


def _encoder_body(xc_ref, xt_ref, W0_ref, b0_ref, We2_ref, be2_ref,
                  Wek_ref, Wl2_ref, bl2_ref, out_ref, *, n, em, depth):
    f32 = jnp.float32
    We2 = We2_ref[...]
    be2 = be2_ref[...]

    def dmat(g):
        x0c = xc_ref[g, :, 0:1]
        x1c = xc_ref[g, :, 1:2]
        x0r = xt_ref[g, 0:1, :]
        x1r = xt_ref[g, 1:2, :]
        d2 = (x0c - x0r) ** 2 + (x1c - x1r) ** 2
        return jnp.sqrt(d2 + 1e-12)

    DA = dmat(0)
    DB = dmat(1)
    Wek = Wek_ref[...]

    def _hi(v):
        bits = jax.lax.bitcast_convert_type(v, jnp.uint32)
        return jax.lax.bitcast_convert_type(
            bits & jnp.uint32(0xFFFF0000), f32)

    DhA = _hi(DA)
    DlA = DA - DhA
    DhB = _hi(DB)
    DlB = DB - DhB
    ones_row = jnp.full((1, n), 1.0, f32)
    planes = []
    for i in range(n):
        lhs = jnp.concatenate(
            [DhA[i:i + 1, :], DhA[i:i + 1, :],
             DlA[i:i + 1, :], DlA[i:i + 1, :],
             DhB[i:i + 1, :], DhB[i:i + 1, :],
             DlB[i:i + 1, :], DlB[i:i + 1, :],
             ones_row, ones_row], axis=0)
        planes.append(jax.lax.dot_general(
            lhs, Wek, (((0,), (0,)), ((), ())),
            preferred_element_type=f32)[None])
    pre = jnp.concatenate(planes, axis=0)
    E = _silu(pre)
    sdiag = _silu(1e-6 * We2 + be2)

    W0r0 = W0_ref[0:1, :]
    W0r1 = W0_ref[1:2, :]
    hA = _silu(xc_ref[0, :, 0:1] * W0r0 + xc_ref[0, :, 1:2] * W0r1
               + b0_ref[...])
    hB = _silu(xc_ref[1, :, 0:1] * W0r0 + xc_ref[1, :, 1:2] * W0r1
               + b0_ref[...])
    h = jnp.concatenate([hA, hB], axis=1)
    for l in range(depth):
        agg = jnp.sum(E * h[:, None, :], axis=0) - sdiag * h
        Wl1 = Wl2_ref[l, :em, :em]
        upd = jnp.concatenate(
            [jnp.dot(agg[:, :em], Wl1, preferred_element_type=f32),
             jnp.dot(agg[:, em:], Wl1, preferred_element_type=f32)],
            axis=1)
        h = _silu(upd + bl2_ref[l]) + h
    out_ref[0] = jnp.sum(h, axis=0, keepdims=True) * (1.0 / n)


@jax.jit
def kernel(x, W0, b0, We, be, Wl, bl):
    seq_len, batch, n, nf = x.shape
    G = seq_len * batch
    em = W0.shape[1]
    depth = Wl.shape[0]
    xc = x.reshape(G, n, nf)
    xt = xc.transpose(0, 2, 1)
    b0r = b0.reshape(1, em)
    We2 = jnp.concatenate([We.reshape(1, em)] * 2, axis=1)
    be2 = jnp.concatenate([be.reshape(1, em)] * 2, axis=1)
    Wl2 = jnp.zeros((depth, 2 * em, 2 * em), jnp.float32)
    Wl2 = Wl2.at[:, :em, :em].set(Wl).at[:, em:, em:].set(Wl)
    bl2 = jnp.concatenate([bl, bl], axis=1)
    zer = jnp.zeros((1, em), jnp.float32)

    def _hi32(v):
        bits = jax.lax.bitcast_convert_type(v, jnp.uint32)
        return jax.lax.bitcast_convert_type(
            bits & jnp.uint32(0xFFFF0000), jnp.float32)

    Wer = We.reshape(1, em)
    Weh = _hi32(Wer)
    Wel = Wer - Weh
    beh2 = _hi32(be2)
    bel2 = be2 - beh2
    WhL = jnp.concatenate([Weh, zer], axis=1)
    WlL = jnp.concatenate([Wel, zer], axis=1)
    WhR = jnp.concatenate([zer, Weh], axis=1)
    WlR = jnp.concatenate([zer, Wel], axis=1)
    Wek = jnp.concatenate([WhL, WlL, WhL, WlL, WhR, WlR, WhR, WlR,
                           beh2, bel2], axis=0)

    body = functools.partial(_encoder_body, n=n, em=em, depth=depth)
    out = pl.pallas_call(
        body,
        grid=(G // 2,),
        in_specs=[
            pl.BlockSpec((2, n, nf), lambda g: (g, 0, 0)),
            pl.BlockSpec((2, nf, n), lambda g: (g, 0, 0)),
            pl.BlockSpec((nf, em), lambda g: (0, 0)),
            pl.BlockSpec((1, em), lambda g: (0, 0)),
            pl.BlockSpec((1, 2 * em), lambda g: (0, 0)),
            pl.BlockSpec((1, 2 * em), lambda g: (0, 0)),
            pl.BlockSpec((10, 2 * em), lambda g: (0, 0)),
            pl.BlockSpec((depth, 2 * em, 2 * em), lambda g: (0, 0, 0)),
            pl.BlockSpec((depth, 2 * em), lambda g: (0, 0)),
        ],
        out_specs=pl.BlockSpec((1, 1, 2 * em), lambda g: (g, 0, 0)),
        out_shape=jax.ShapeDtypeStruct((G // 2, 1, 2 * em), jnp.float32),
    )(xc, xt, W0, b0r, We2, be2, Wek, Wl2, bl2)
    return out.reshape(seq_len, batch, em)


# speedup vs baseline: 1.0741x; 1.0741x over previous
import functools

import jax
import jax.numpy as jnp
from jax.experimental import pallas as pl
from jax.experimental.pallas import tpu as pltpu

_DEPTH = 6


def _silu(x):
    return (0.5 * x) * (1.0 + jnp.tanh(0.5 * x))


def _encoder_body(xc_ref, xt_ref, W0_ref, b0_ref, We2_ref, be2_ref,
                  Wek_ref, Wl2_ref, bl2_ref, out_ref, *, n, em, depth):
    f32 = jnp.float32
    We2 = We2_ref[...]
    be2 = be2_ref[...]

    def dmat(g):
        x0c = xc_ref[g, :, 0:1]
        x1c = xc_ref[g, :, 1:2]
        x0r = xt_ref[g, 0:1, :]
        x1r = xt_ref[g, 1:2, :]
        d2 = (x0c - x0r) ** 2 + (x1c - x1r) ** 2
        return jnp.sqrt(d2 + 1e-12)

    DA = dmat(0)
    DB = dmat(1)
    Wek = Wek_ref[...]

    def _hi(v):
        bits = jax.lax.bitcast_convert_type(v, jnp.uint32)
        return jax.lax.bitcast_convert_type(
            bits & jnp.uint32(0xFFFF0000), f32)

    DhA = _hi(DA)
    DlA = DA - DhA
    DhB = _hi(DB)
    DlB = DB - DhB
    ones_row = jnp.full((1, n), 1.0, f32)
    planes = []
    for i in range(n):
        lhs = jnp.concatenate(
            [DhA[i:i + 1, :], DhA[i:i + 1, :],
             DlA[i:i + 1, :], DlA[i:i + 1, :],
             DhB[i:i + 1, :], DhB[i:i + 1, :],
             DlB[i:i + 1, :], DlB[i:i + 1, :],
             ones_row, ones_row], axis=0)
        planes.append(jax.lax.dot_general(
            lhs, Wek, (((0,), (0,)), ((), ())),
            preferred_element_type=f32)[None])
    pre = jnp.concatenate(planes, axis=0)
    E = _silu(pre)
    sdiag = _silu(1e-6 * We2 + be2)

    W0r0 = W0_ref[0:1, :]
    W0r1 = W0_ref[1:2, :]
    hA = _silu(xc_ref[0, :, 0:1] * W0r0 + xc_ref[0, :, 1:2] * W0r1
               + b0_ref[...])
    hB = _silu(xc_ref[1, :, 0:1] * W0r0 + xc_ref[1, :, 1:2] * W0r1
               + b0_ref[...])
    h = jnp.concatenate([hA, hB], axis=1)
    for l in range(depth):
        agg = jnp.sum(E * h[:, None, :], axis=0) - sdiag * h
        Wl1 = Wl2_ref[l, :em, :em]
        upd = jnp.concatenate(
            [jnp.dot(agg[:, :em], Wl1, preferred_element_type=f32),
             jnp.dot(agg[:, em:], Wl1, preferred_element_type=f32)],
            axis=1)
        h = _silu(upd + bl2_ref[l]) + h
    out_ref[0] = jnp.sum(h, axis=0, keepdims=True) * (1.0 / n)


@jax.jit
def kernel(x, W0, b0, We, be, Wl, bl):
    seq_len, batch, n, nf = x.shape
    G = seq_len * batch
    em = W0.shape[1]
    depth = Wl.shape[0]
    xc = x.reshape(G, n, nf)
    xt = xc.transpose(0, 2, 1)
    b0r = b0.reshape(1, em)
    We2 = jnp.concatenate([We.reshape(1, em)] * 2, axis=1)
    be2 = jnp.concatenate([be.reshape(1, em)] * 2, axis=1)
    Wl2 = jnp.zeros((depth, 2 * em, 2 * em), jnp.float32)
    Wl2 = Wl2.at[:, :em, :em].set(Wl).at[:, em:, em:].set(Wl)
    bl2 = jnp.concatenate([bl, bl], axis=1)
    zer = jnp.zeros((1, em), jnp.float32)

    def _hi32(v):
        bits = jax.lax.bitcast_convert_type(v, jnp.uint32)
        return jax.lax.bitcast_convert_type(
            bits & jnp.uint32(0xFFFF0000), jnp.float32)

    Wer = We.reshape(1, em)
    Weh = _hi32(Wer)
    Wel = Wer - Weh
    beh2 = _hi32(be2)
    bel2 = be2 - beh2
    WhL = jnp.concatenate([Weh, zer], axis=1)
    WlL = jnp.concatenate([Wel, zer], axis=1)
    WhR = jnp.concatenate([zer, Weh], axis=1)
    WlR = jnp.concatenate([zer, Wel], axis=1)
    Wek = jnp.concatenate([WhL, WlL, WhL, WlL, WhR, WlR, WhR, WlR,
                           beh2, bel2], axis=0)

    body = functools.partial(_encoder_body, n=n, em=em, depth=depth)
    out = pl.pallas_call(
        body,
        grid=(G // 2,),
        in_specs=[
            pl.BlockSpec((2, n, nf), lambda g: (g, 0, 0)),
            pl.BlockSpec((2, nf, n), lambda g: (g, 0, 0)),
            pl.BlockSpec((nf, em), lambda g: (0, 0)),
            pl.BlockSpec((1, em), lambda g: (0, 0)),
            pl.BlockSpec((1, 2 * em), lambda g: (0, 0)),
            pl.BlockSpec((1, 2 * em), lambda g: (0, 0)),
            pl.BlockSpec((10, 2 * em), lambda g: (0, 0)),
            pl.BlockSpec((depth, 2 * em, 2 * em), lambda g: (0, 0, 0)),
            pl.BlockSpec((depth, 2 * em), lambda g: (0, 0)),
        ],
        out_specs=pl.BlockSpec((1, 1, 2 * em), lambda g: (g, 0, 0)),
        out_shape=jax.ShapeDtypeStruct((G // 2, 1, 2 * em), jnp.float32),
    )(xc, xt, W0, b0r, We2, be2, Wek, Wl2, bl2)
    return out.reshape(seq_len, batch, em)


# speedup vs baseline: 1.2946x; 1.2052x over previous
import functools

import jax
import jax.numpy as jnp
from jax.experimental import pallas as pl
from jax.experimental.pallas import tpu as pltpu

_DEPTH = 6
_PACK = 4


def _silu(x):
    return (0.5 * x) * (1.0 + jnp.tanh(0.5 * x))


def _hi(v):
    bits = jax.lax.bitcast_convert_type(v, jnp.uint32)
    return jax.lax.bitcast_convert_type(
        bits & jnp.uint32(0xFFFF0000), jnp.float32)


def _encoder_body(xc_ref, xt_ref, W0_ref, b0_ref, WeP_ref, beP_ref,
                  Wek_ref, Wl_ref, blP_ref, out_ref, *, n, em, depth, p):
    f32 = jnp.float32
    WeP = WeP_ref[...]
    beP = beP_ref[...]

    def dmat(g):
        x0c = xc_ref[g, :, 0:1]
        x1c = xc_ref[g, :, 1:2]
        x0r = xt_ref[g, 0:1, :]
        x1r = xt_ref[g, 1:2, :]
        d2 = (x0c - x0r) ** 2 + (x1c - x1r) ** 2
        return jnp.sqrt(d2 + 1e-12)

    Dh, Dl = [], []
    for g in range(p):
        D = dmat(g)
        h_part = _hi(D)
        Dh.append(h_part)
        Dl.append(D - h_part)

    Wek = Wek_ref[...]
    ones_row = jnp.full((1, n), 1.0, f32)
    planes = []
    for i in range(n):
        rows = []
        for g in range(p):
            rows += [Dh[g][i:i + 1, :], Dh[g][i:i + 1, :],
                     Dl[g][i:i + 1, :], Dl[g][i:i + 1, :]]
        rows += [ones_row, ones_row]
        lhs = jnp.concatenate(rows, axis=0)
        planes.append(jax.lax.dot_general(
            lhs, Wek, (((0,), (0,)), ((), ())),
            preferred_element_type=f32)[None])
    pre = jnp.concatenate(planes, axis=0)
    E = _silu(pre)
    sdiag = _silu(1e-6 * WeP + beP)

    W0r0 = W0_ref[0:1, :]
    W0r1 = W0_ref[1:2, :]
    b0r = b0_ref[...]
    h = jnp.concatenate(
        [_silu(xc_ref[g, :, 0:1] * W0r0 + xc_ref[g, :, 1:2] * W0r1 + b0r)
         for g in range(p)], axis=1)
    for l in range(depth):
        agg = jnp.sum(E * h[:, None, :], axis=0) - sdiag * h
        Wl1 = Wl_ref[l]
        upd = jnp.concatenate(
            [jnp.dot(agg[:, g * em:(g + 1) * em], Wl1,
                     preferred_element_type=f32) for g in range(p)],
            axis=1)
        h = _silu(upd + blP_ref[l]) + h
    out_ref[0] = jnp.sum(h, axis=0, keepdims=True) * (1.0 / n)


@jax.jit
def kernel(x, W0, b0, We, be, Wl, bl):
    seq_len, batch, n, nf = x.shape
    G = seq_len * batch
    em = W0.shape[1]
    depth = Wl.shape[0]
    p = _PACK
    xc = x.reshape(G, n, nf)
    xt = xc.transpose(0, 2, 1)
    b0r = b0.reshape(1, em)
    Wer = We.reshape(1, em)
    ber = be.reshape(1, em)
    WeP = jnp.concatenate([Wer] * p, axis=1)
    beP = jnp.concatenate([ber] * p, axis=1)
    blP = jnp.concatenate([bl] * p, axis=1)

    Weh = _hi(Wer)
    Wel = Wer - Weh
    behP = _hi(beP)
    belP = beP - behP
    zer = jnp.zeros((1, em), jnp.float32)

    def block(row, g):
        return jnp.concatenate([zer] * g + [row] + [zer] * (p - 1 - g),
                               axis=1)

    wrows = []
    for g in range(p):
        wrows += [block(Weh, g), block(Wel, g), block(Weh, g), block(Wel, g)]
    wrows += [behP, belP]
    Wek = jnp.concatenate(wrows, axis=0)

    body = functools.partial(_encoder_body, n=n, em=em, depth=depth, p=p)
    out = pl.pallas_call(
        body,
        grid=(G // p,),
        in_specs=[
            pl.BlockSpec((p, n, nf), lambda g: (g, 0, 0)),
            pl.BlockSpec((p, nf, n), lambda g: (g, 0, 0)),
            pl.BlockSpec((nf, em), lambda g: (0, 0)),
            pl.BlockSpec((1, em), lambda g: (0, 0)),
            pl.BlockSpec((1, p * em), lambda g: (0, 0)),
            pl.BlockSpec((1, p * em), lambda g: (0, 0)),
            pl.BlockSpec((4 * p + 2, p * em), lambda g: (0, 0)),
            pl.BlockSpec((depth, em, em), lambda g: (0, 0, 0)),
            pl.BlockSpec((depth, p * em), lambda g: (0, 0)),
        ],
        out_specs=pl.BlockSpec((1, 1, p * em), lambda g: (g, 0, 0)),
        out_shape=jax.ShapeDtypeStruct((G // p, 1, p * em), jnp.float32),
    )(xc, xt, W0, b0r, WeP, beP, Wek, Wl, blP)
    return out.reshape(seq_len, batch, em)
